# baseline (speedup 1.0000x reference)
import functools

import jax
import jax.numpy as jnp
from jax import lax
from jax.experimental import pallas as pl
from jax.experimental.pallas import tpu as pltpu
from jax.experimental.pallas import tpu_sc as plsc

_B, _S = 64, 8192
_NC, _NS, _L = 2, 16, 16
_NW = _NC * _NS
_ROWS_PER_W = _B // _NW
_NCHUNK = _S // _L
_MAX_ITERS = 16
_NEG = -1e30


def _scalar_sum16(vec):
    total = vec[0]
    for j in range(1, _L):
        total = total + vec[j]
    return total


def _scalar_max16(vec):
    best = vec[0]
    for j in range(1, _L):
        best = jnp.maximum(best, vec[j])
    return best


def _sc_body(scores_hbm, maskf_hbm, out_hbm, z_ref, m_ref):
    wid = lax.axis_index("s") * _NC + lax.axis_index("c")
    zeros = jnp.zeros((_L,), jnp.float32)

    for r in range(_ROWS_PER_W):
        row = wid * _ROWS_PER_W + r
        pltpu.sync_copy(scores_hbm.at[row], z_ref)
        pltpu.sync_copy(maskf_hbm.at[row], m_ref)

        def p1(i, carry):
            sacc, macc = carry
            v = z_ref[pl.ds(i * _L, _L)]
            m = m_ref[pl.ds(i * _L, _L)]
            z = jnp.where(m != 0.0, v, jnp.float32(_NEG))
            z_ref[pl.ds(i * _L, _L)] = z
            return sacc + z, jnp.maximum(macc, z)

        sacc, macc = lax.fori_loop(
            0, _NCHUNK, p1, (zeros, jnp.full((_L,), _NEG, jnp.float32)))
        total = _scalar_sum16(sacc)
        mx = _scalar_max16(macc)
        mx_vec = jnp.broadcast_to(mx, (_L,))
        tau0 = jnp.minimum(
            jnp.broadcast_to((total - 1.0) * jnp.float32(1.0 / _S), (_L,)),
            mx_vec)

        def newton_body(_, tau):
            def scan(i, c):
                s_acc, k_acc = c
                z = z_ref[pl.ds(i * _L, _L)]
                sel = z >= tau
                return (s_acc + jnp.where(sel, z, 0.0),
                        k_acc + jnp.where(sel, 1.0, 0.0))

            s_acc, k_acc = lax.fori_loop(0, _NCHUNK, scan, (zeros, zeros))
            num = jnp.broadcast_to(_scalar_sum16(s_acc) - 1.0, (_L,))
            den = jnp.broadcast_to(
                jnp.maximum(_scalar_sum16(k_acc), 1.0), (_L,))
            return jnp.minimum(num / den, mx_vec)

        tau = lax.fori_loop(0, _MAX_ITERS, newton_body, tau0)

        def outp(i, carry):
            z = z_ref[pl.ds(i * _L, _L)]
            z_ref[pl.ds(i * _L, _L)] = jnp.maximum(z - tau, 0.0)
            return carry

        lax.fori_loop(0, _NCHUNK, outp, 0)
        pltpu.sync_copy(z_ref, out_hbm.at[row])


@jax.jit
def _sc_sparsemax(scores, maskf):
    mesh = plsc.VectorSubcoreMesh(core_axis_name="c", subcore_axis_name="s")
    return pl.kernel(
        _sc_body,
        mesh=mesh,
        out_type=jax.ShapeDtypeStruct((_B, _S), jnp.float32),
        scratch_types=[
            pltpu.VMEM((_S,), jnp.float32),
            pltpu.VMEM((_S,), jnp.float32),
        ],
    )(scores, maskf)


def kernel(scores, mask):
    return _sc_sparsemax(scores, mask.astype(jnp.float32))


# speedup vs baseline: 1.8456x; 1.8456x over previous
import functools

import jax
import jax.numpy as jnp
from jax import lax
from jax.experimental import pallas as pl
from jax.experimental.pallas import tpu as pltpu
from jax.experimental.pallas import tpu_sc as plsc

_B, _S = 64, 8192
_NC, _NS, _L = 2, 16, 16
_NW = _NC * _NS
_ROWS_PER_W = _B // _NW
_NCHUNK = _S // _L
_UF = 4
_MAX_ITERS = 16
_NEG = -1e30


def _scalar_sum16(vec):
    total = vec[0]
    for j in range(1, _L):
        total = total + vec[j]
    return total


def _scalar_max16(vec):
    best = vec[0]
    for j in range(1, _L):
        best = jnp.maximum(best, vec[j])
    return best


def _sc_body(scores_hbm, maskf_hbm, out_hbm, z_ref, m_ref):
    wid = lax.axis_index("s") * _NC + lax.axis_index("c")
    zeros = jnp.zeros((_L,), jnp.float32)
    negs = jnp.full((_L,), _NEG, jnp.float32)

    for r in range(_ROWS_PER_W):
        row = wid * _ROWS_PER_W + r
        pltpu.sync_copy(scores_hbm.at[row], z_ref)
        pltpu.sync_copy(maskf_hbm.at[row], m_ref)

        def p1(i, carry):
            accs = list(carry)
            base = i * (_UF * _L)
            for u in range(_UF):
                v = z_ref[pl.ds(base + u * _L, _L)]
                m = m_ref[pl.ds(base + u * _L, _L)]
                z = jnp.where(m != 0.0, v, jnp.float32(_NEG))
                z_ref[pl.ds(base + u * _L, _L)] = z
                accs[u] = accs[u] + z
                accs[_UF + u] = jnp.maximum(accs[_UF + u], z)
            return tuple(accs)

        accs = lax.fori_loop(0, _NCHUNK // _UF, p1,
                             (zeros,) * _UF + (negs,) * _UF)
        sacc = accs[0] + accs[1] + accs[2] + accs[3]
        macc = jnp.maximum(jnp.maximum(accs[4], accs[5]),
                           jnp.maximum(accs[6], accs[7]))
        total = _scalar_sum16(sacc)
        mx = _scalar_max16(macc)
        mx_vec = jnp.broadcast_to(mx, (_L,))
        tau0 = jnp.minimum(
            jnp.broadcast_to((total - 1.0) * jnp.float32(1.0 / _S), (_L,)),
            mx_vec)

        def newton_body(_, tau):
            def scan(i, c):
                accs = list(c)
                base = i * (_UF * _L)
                for u in range(_UF):
                    z = z_ref[pl.ds(base + u * _L, _L)]
                    sel = z >= tau
                    accs[u] = accs[u] + jnp.where(sel, z, 0.0)
                    accs[_UF + u] = accs[_UF + u] + jnp.where(sel, 1.0, 0.0)
                return tuple(accs)

            accs = lax.fori_loop(0, _NCHUNK // _UF, scan, (zeros,) * (2 * _UF))
            s_acc = accs[0] + accs[1] + accs[2] + accs[3]
            k_acc = accs[4] + accs[5] + accs[6] + accs[7]
            num = jnp.broadcast_to(_scalar_sum16(s_acc) - 1.0, (_L,))
            den = jnp.broadcast_to(
                jnp.maximum(_scalar_sum16(k_acc), 1.0), (_L,))
            return jnp.minimum(num / den, mx_vec)

        tau = lax.fori_loop(0, _MAX_ITERS, newton_body, tau0)

        def outp(i, carry):
            base = i * (_UF * _L)
            for u in range(_UF):
                z = z_ref[pl.ds(base + u * _L, _L)]
                z_ref[pl.ds(base + u * _L, _L)] = jnp.maximum(z - tau, 0.0)
            return carry

        lax.fori_loop(0, _NCHUNK // _UF, outp, 0)
        pltpu.sync_copy(z_ref, out_hbm.at[row])


@jax.jit
def _sc_sparsemax(scores, maskf):
    mesh = plsc.VectorSubcoreMesh(core_axis_name="c", subcore_axis_name="s")
    return pl.kernel(
        _sc_body,
        mesh=mesh,
        out_type=jax.ShapeDtypeStruct((_B, _S), jnp.float32),
        scratch_types=[
            pltpu.VMEM((_S,), jnp.float32),
            pltpu.VMEM((_S,), jnp.float32),
        ],
    )(scores, maskf)


def kernel(scores, mask):
    return _sc_sparsemax(scores, mask.astype(jnp.float32))
